# Initial kernel scaffold; baseline (speedup 1.0000x reference)
import jax
import jax.numpy as jnp
from jax.experimental import pallas as pl


def kernel(x, img_dim):
    raise NotImplementedError("write your pallas kernel here")


# speedup vs baseline: 3.1808x; 3.1808x over previous
import jax
import jax.numpy as jnp
from jax.experimental import pallas as pl
from jax.experimental.pallas import tpu as pltpu

_ANCHORS = ((116.0, 90.0), (156.0, 198.0), (373.0, 326.0))
_NUM_ANCHORS = 3
_NUM_CLASSES = 80
_C = _NUM_CLASSES + 5


def _decode_kernel(params_ref, x_ref, o_ref, *, g: int):
    cells = g * g
    a = pl.program_id(0) % _NUM_ANCHORS
    stride = params_ref[0, 0]
    aw = params_ref[1, a]
    ah = params_ref[2, a]

    v = x_ref[0]
    sig = jax.nn.sigmoid(v)

    top = v[0:8, :]
    sig_top = sig[0:8, :]
    exp_top = jnp.exp(top)
    r = jax.lax.broadcasted_iota(jnp.int32, (8, cells), 0)
    kf = jax.lax.broadcasted_iota(jnp.int32, (8, cells), 1).astype(jnp.float32)
    gy = jnp.floor((kf + 0.5) * (1.0 / g))
    gx = kf - g * gy
    box = jnp.where(
        r == 0, (sig_top + gx) * stride,
        jnp.where(
            r == 1, (sig_top + gy) * stride,
            jnp.where(r == 2, exp_top * aw,
                      jnp.where(r == 3, exp_top * ah, sig_top))))
    res = jnp.concatenate([box, sig[8:, :]], axis=0)
    o_ref[0] = res.T


def kernel(x, img_dim):
    n, c_all, g, _ = x.shape
    cells = g * g
    x3 = x.reshape(n * _NUM_ANCHORS, _C, cells)

    stride = (jnp.asarray(img_dim, jnp.float32) / g)
    anchors = jnp.array(_ANCHORS, jnp.float32)
    params = jnp.stack(
        [jnp.full((_NUM_ANCHORS,), stride), anchors[:, 0], anchors[:, 1]],
        axis=0)

    import functools
    out = pl.pallas_call(
        functools.partial(_decode_kernel, g=g),
        grid=(n * _NUM_ANCHORS,),
        in_specs=[
            pl.BlockSpec(memory_space=pltpu.SMEM),
            pl.BlockSpec((1, _C, cells), lambda i: (i, 0, 0)),
        ],
        out_specs=pl.BlockSpec((1, cells, _C), lambda i: (i, 0, 0)),
        out_shape=jax.ShapeDtypeStruct((n * _NUM_ANCHORS, cells, _C),
                                       jnp.float32),
    )(params, x3)
    return out.reshape(n, _NUM_ANCHORS * cells, _C)
